# baseline (speedup 1.0000x reference)
import functools

import jax
import jax.numpy as jnp
from jax import lax
from jax.experimental import pallas as pl
from jax.experimental.pallas import tpu as pltpu
from jax.experimental.pallas import tpu_sc as plsc

T, B, D = 128, 4096, 64
M = 65536
NC, NS, L = 2, 16, 16
NW = NC * NS
RPW = M // NW
CHUNK = 128
NCHUNK = RPW // CHUNK
GPC = CHUNK // L


def _sc_gather(mem_flat, val, steplo, idx2d):
    mesh = plsc.VectorSubcoreMesh(core_axis_name="c", subcore_axis_name="s")

    @functools.partial(
        pl.kernel,
        mesh=mesh,
        compiler_params=pltpu.CompilerParams(
            use_tc_tiling_on_sc=False, needs_layout_passes=False),
        out_type=jax.ShapeDtypeStruct((M, D), jnp.float32),
        scratch_types=[
            pltpu.VMEM((NCHUNK, CHUNK), jnp.int32),
            pltpu.VMEM((CHUNK, D), jnp.float32),
            pltpu.VMEM((L, D), jnp.float32),
            pltpu.VMEM((L,), jnp.int32),
            pltpu.SemaphoreType.DMA,
            pltpu.SemaphoreType.DMA,
        ],
    )
    def k(mem_hbm, val_hbm, steplo_hbm, idx_hbm, out_hbm,
          idx_v, buf_v, valbuf_v, steplo_v, gsem, vsem):
        wid = lax.axis_index("s") * NC + lax.axis_index("c")
        base = wid * RPW
        pltpu.sync_copy(idx_hbm.at[pl.ds(wid * NCHUNK, NCHUNK)], idx_v)
        pltpu.sync_copy(steplo_hbm, steplo_v)
        steplo = steplo_v[...]
        stephi = steplo + B
        lane = lax.iota(jnp.int32, L)

        def chunk_body(c, carry):
            pltpu.async_copy(mem_hbm.at[idx_v.at[c]], buf_v, gsem).wait()
            for g in range(GPC):
                idxg = idx_v[c, pl.ds(g * L, L)]
                mask = (idxg >= steplo) & (idxg < stephi)
                cnt = jnp.sum(mask.astype(jnp.int32))

                @pl.when(cnt > 0)
                def _fix():
                    validx = jnp.where(mask, idxg - steplo, 0)
                    pltpu.async_copy(val_hbm.at[validx], valbuf_v, vsem).wait()
                    rows = jnp.full((L,), g * L, jnp.int32) + lane

                    def col_body(j, c2):
                        jv = jnp.full((L,), j, jnp.int32)
                        v = plsc.load_gather(valbuf_v, [lane, jv])
                        plsc.store_scatter(buf_v, [rows, jv], v, mask=mask)
                        return c2

                    lax.fori_loop(0, D, col_body, 0)

            pltpu.sync_copy(buf_v, out_hbm.at[pl.ds(base + c * CHUNK, CHUNK)])
            return carry

        lax.fori_loop(0, NCHUNK, chunk_body, 0)

    return k(mem_flat, val, steplo, idx2d)


def kernel(mem, val, step, batch_idx):
    mem_flat = mem.reshape(T * B, D)
    steplo = jnp.full((L,), jnp.int32(step) * B, dtype=jnp.int32)
    idx2d = batch_idx.reshape(M // CHUNK, CHUNK)
    return _sc_gather(mem_flat, val, steplo, idx2d)
